# Initial kernel scaffold; baseline (speedup 1.0000x reference)
import functools

import jax
import jax.numpy as jnp
from jax import lax
from jax.experimental import pallas as pl
from jax.experimental.pallas import tpu as pltpu
from jax.experimental.pallas import tpu_sc as plsc

N_NODES = 10000
N_EDGES = 320000
D_ATOM = 128
D_BOND = 16
D_HID = 128
EPS = 1e-5

NC = 2
NS = 16
NW = NC * NS
SCAT = 80
N_CHUNKS = N_EDGES // SCAT
CPW = N_CHUNKS // NW
STAGE = 25
NSTAGE = CPW // STAGE
ROWS_PER_TILE = N_NODES // NS


def _sc_body(ea_hbm, idx_hbm, acc_out, cnt_out,
             ea_v, idx_v, ones_v, zrow_v, zcnt_v, acc_sh, cnt_sh):
  c = lax.axis_index("c")
  s = lax.axis_index("s")
  w = c * NS + s

  z16 = jnp.zeros((16,), jnp.float32)
  one16 = jnp.ones((16,), jnp.float32)

  for j in range(SCAT // 16):
    ones_v[pl.ds(j * 16, 16)] = one16

  def zrow_fill(i, carry):
    zrow_v[i, :] = z16
    return carry
  lax.fori_loop(0, ROWS_PER_TILE, zrow_fill, 0)

  def zcnt_fill(i, carry):
    zcnt_v[pl.ds(i * 16, 16)] = z16
    return carry
  lax.fori_loop(0, N_NODES // 16, zcnt_fill, 0)

  pltpu.sync_copy(zrow_v, acc_sh.at[pl.ds(s * ROWS_PER_TILE, ROWS_PER_TILE)])

  @pl.when(s == 0)
  def _():
    pltpu.sync_copy(zcnt_v, cnt_sh)

  plsc.subcore_barrier()

  def stage_body(t, carry):
    off = w * CPW + t * STAGE
    pltpu.sync_copy(ea_hbm.at[pl.ds(off, STAGE)], ea_v)
    pltpu.sync_copy(idx_hbm.at[pl.ds(off, STAGE)], idx_v)
    for j in range(STAGE):
      pltpu.sync_copy(ea_v.at[j], acc_sh.at[idx_v.at[j]], add=True)
      pltpu.sync_copy(ones_v, cnt_sh.at[idx_v.at[j]], add=True)
    return carry

  lax.fori_loop(0, NSTAGE, stage_body, 0)

  plsc.subcore_barrier()

  @pl.when(s == 0)
  def _():
    pltpu.sync_copy(acc_sh, acc_out.at[c])
    pltpu.sync_copy(cnt_sh, cnt_out.at[c])


_sc_segment_sum = functools.partial(
    pl.kernel,
    out_type=(
        jax.ShapeDtypeStruct((NC, N_NODES, D_BOND), jnp.float32),
        jax.ShapeDtypeStruct((NC, N_NODES), jnp.float32),
    ),
    mesh=plsc.VectorSubcoreMesh(core_axis_name="c", subcore_axis_name="s",
                                num_cores=NC, num_subcores=NS),
    scratch_types=[
        pltpu.VMEM((STAGE, SCAT, D_BOND), jnp.float32),
        pltpu.VMEM((STAGE, SCAT), jnp.int32),
        pltpu.VMEM((SCAT,), jnp.float32),
        pltpu.VMEM((ROWS_PER_TILE, D_BOND), jnp.float32),
        pltpu.VMEM((N_NODES,), jnp.float32),
        pltpu.VMEM_SHARED((N_NODES, D_BOND), jnp.float32),
        pltpu.VMEM_SHARED((N_NODES,), jnp.float32),
    ],
)(_sc_body)


def _tc_body(p_ref, cnt_ref, x_ref, wa_ref, ba_ref, wb_ref, bb_ref,
             g_ref, be_ref, o_ref):
  agg = p_ref[0] + p_ref[1]
  h = lax.dot_general(x_ref[...], wa_ref[...],
                      (((1,), (1,)), ((), ())),
                      preferred_element_type=jnp.float32)
  h = h + lax.dot_general(agg, wb_ref[...],
                          (((1,), (1,)), ((), ())),
                          preferred_element_type=jnp.float32)
  bb2 = jnp.broadcast_to(bb_ref[...], (2, D_HID))
  h = h + lax.dot_general(cnt_ref[...], bb2,
                          (((1,), (0,)), ((), ())),
                          preferred_element_type=jnp.float32)
  h = h + ba_ref[...]
  h = jnp.maximum(h, 0.0)
  mean = jnp.mean(h, axis=0, keepdims=True)
  d = h - mean
  var = jnp.mean(d * d, axis=0, keepdims=True)
  o_ref[...] = g_ref[...] * (d * lax.rsqrt(var + EPS)) + be_ref[...]


_tc_dense = pl.pallas_call(
    _tc_body,
    out_shape=jax.ShapeDtypeStruct((N_NODES, D_HID), jnp.float32),
)


def kernel(x, edge_index, edge_attr, W_atom, b_atom, W_bond, b_bond,
           gamma, beta):
  dst = edge_index[1].astype(jnp.int32)
  ea3 = edge_attr.reshape(N_CHUNKS, SCAT, D_BOND)
  idx2 = dst.reshape(N_CHUNKS, SCAT)
  partials, counts = _sc_segment_sum(ea3, idx2)
  return _tc_dense(partials, counts.T, x,
                   W_atom, b_atom.reshape(1, D_HID),
                   W_bond, b_bond.reshape(1, D_HID),
                   gamma.reshape(1, D_HID), beta.reshape(1, D_HID))


# speedup vs baseline: 5.4692x; 5.4692x over previous
import functools

import jax
import jax.numpy as jnp
from jax import lax
from jax.experimental import pallas as pl
from jax.experimental.pallas import tpu as pltpu
from jax.experimental.pallas import tpu_sc as plsc

N_NODES = 10000
N_EDGES = 320000
D_ATOM = 128
D_BOND = 16
D_HID = 128
EPS = 1e-5

NC = 2
NS = 16
NW = NC * NS
EPW = N_EDGES // NW
SCAT = 125
WPW = EPW // SCAT
STAGE = 2000
NSTAGE = EPW // STAGE
WPS = STAGE // SCAT
ROWS_PER_TILE = N_NODES // NS


def _sc_body(ea_hbm, idx_hbm, acc_out, cnt0_out, cnt1_out,
             ea_v, idx_v, ones_v, zrow_v, zcnt_v, acc_sh, cnt_sh):
  c = lax.axis_index("c")
  s = lax.axis_index("s")
  w = c * NS + s

  z16 = jnp.zeros((16,), jnp.float32)
  one16 = jnp.ones((16,), jnp.float32)

  for j in range(128 // 16):
    ones_v[pl.ds(j * 16, 16)] = one16

  def zrow_fill(i, carry):
    zrow_v[i, :] = z16
    return carry
  lax.fori_loop(0, ROWS_PER_TILE, zrow_fill, 0)

  pltpu.sync_copy(zrow_v, acc_sh.at[pl.ds(s * ROWS_PER_TILE, ROWS_PER_TILE)])

  @pl.when(s == 0)
  def _():
    def zcnt_fill(i, carry):
      zcnt_v[pl.ds(i * 16, 16)] = z16
      return carry
    lax.fori_loop(0, N_NODES // 16, zcnt_fill, 0)
    pltpu.sync_copy(zcnt_v, cnt_sh)

  pltpu.sync_copy(idx_hbm.at[w], idx_v)

  plsc.subcore_barrier()

  ea_w = ea_hbm.at[w]

  def stage_body(t, carry):
    pltpu.sync_copy(ea_w.at[pl.ds(t * STAGE, STAGE)], ea_v)
    for j in range(WPS):
      idx_row = idx_v.at[t * WPS + j]
      pltpu.sync_copy(ea_v.at[pl.ds(j * SCAT, SCAT)],
                      acc_sh.at[idx_row], add=True)
      pltpu.sync_copy(ones_v.at[pl.ds(0, SCAT)],
                      cnt_sh.at[idx_row], add=True)
    return carry

  lax.fori_loop(0, NSTAGE, stage_body, 0)

  plsc.subcore_barrier()

  @pl.when(s == 0)
  def _():
    pltpu.sync_copy(acc_sh, acc_out.at[c])

  @pl.when(jnp.logical_and(s == 0, c == 0))
  def _():
    pltpu.sync_copy(cnt_sh, cnt0_out)

  @pl.when(jnp.logical_and(s == 0, c == 1))
  def _():
    pltpu.sync_copy(cnt_sh, cnt1_out)


_sc_segment_sum = functools.partial(
    pl.kernel,
    out_type=(
        jax.ShapeDtypeStruct((NC, N_NODES, D_BOND), jnp.float32),
        jax.ShapeDtypeStruct((N_NODES,), jnp.float32),
        jax.ShapeDtypeStruct((N_NODES,), jnp.float32),
    ),
    mesh=plsc.VectorSubcoreMesh(core_axis_name="c", subcore_axis_name="s",
                                num_cores=NC, num_subcores=NS),
    compiler_params=pltpu.CompilerParams(use_tc_tiling_on_sc=False),
    scratch_types=[
        pltpu.VMEM((STAGE, D_BOND), jnp.float32),
        pltpu.VMEM((WPW, SCAT), jnp.int32),
        pltpu.VMEM((128,), jnp.float32),
        pltpu.VMEM((ROWS_PER_TILE, D_BOND), jnp.float32),
        pltpu.VMEM((N_NODES,), jnp.float32),
        pltpu.VMEM_SHARED((N_NODES, D_BOND), jnp.float32),
        pltpu.VMEM_SHARED((N_NODES,), jnp.float32),
    ],
)(_sc_body)


def _tc_body(p_ref, cnt_ref, x_ref, wa_ref, ba_ref, wb_ref, bb_ref,
             g_ref, be_ref, o_ref):
  agg = p_ref[0] + p_ref[1]
  h = lax.dot_general(x_ref[...], wa_ref[...],
                      (((1,), (1,)), ((), ())),
                      preferred_element_type=jnp.float32)
  h = h + lax.dot_general(agg, wb_ref[...],
                          (((1,), (1,)), ((), ())),
                          preferred_element_type=jnp.float32)
  bb2 = jnp.broadcast_to(bb_ref[...], (2, D_HID))
  h = h + lax.dot_general(cnt_ref[...], bb2,
                          (((1,), (0,)), ((), ())),
                          preferred_element_type=jnp.float32)
  h = h + ba_ref[...]
  h = jnp.maximum(h, 0.0)
  mean = jnp.mean(h, axis=0, keepdims=True)
  d = h - mean
  var = jnp.mean(d * d, axis=0, keepdims=True)
  o_ref[...] = g_ref[...] * (d * lax.rsqrt(var + EPS)) + be_ref[...]


_tc_dense = pl.pallas_call(
    _tc_body,
    out_shape=jax.ShapeDtypeStruct((N_NODES, D_HID), jnp.float32),
)


def kernel(x, edge_index, edge_attr, W_atom, b_atom, W_bond, b_bond,
           gamma, beta):
  dst = edge_index[1].astype(jnp.int32)
  idx3 = dst.reshape(NW, WPW, SCAT)
  ea3 = edge_attr.reshape(NW, EPW, D_BOND)
  partials, cnt0, cnt1 = _sc_segment_sum(ea3, idx3)
  counts2 = jnp.stack([cnt0, cnt1], axis=1)
  return _tc_dense(partials, counts2, x,
                   W_atom, b_atom.reshape(1, D_HID),
                   W_bond, b_bond.reshape(1, D_HID),
                   gamma.reshape(1, D_HID), beta.reshape(1, D_HID))


# speedup vs baseline: 5.8801x; 1.0751x over previous
import functools

import jax
import jax.numpy as jnp
from jax import lax
from jax.experimental import pallas as pl
from jax.experimental.pallas import tpu as pltpu
from jax.experimental.pallas import tpu_sc as plsc

N_NODES = 10000
N_EDGES = 320000
D_ATOM = 128
D_BOND = 16
D_HID = 128
EPS = 1e-5

NC = 2
NS = 16
NW = NC * NS
EPW = N_EDGES // NW
SCAT = 125
WPW = EPW // SCAT
STAGE = 2000
NSTAGE = EPW // STAGE
WPS = STAGE // SCAT
ROWS_PER_TILE = N_NODES // NS


def _sc_body(ea_hbm, idx_hbm, acc_out, cnt0_out, cnt1_out,
             ea_v, idx_v, ones_v, zrow_v, zcnt_v, acc_sh, cnt_sh,
             ld_sem, scat_sem):
  c = lax.axis_index("c")
  s = lax.axis_index("s")
  w = c * NS + s

  z16 = jnp.zeros((16,), jnp.float32)
  one16 = jnp.ones((16,), jnp.float32)

  for j in range(128 // 16):
    ones_v[pl.ds(j * 16, 16)] = one16

  def zrow_fill(i, carry):
    zrow_v[i, :] = z16
    return carry
  lax.fori_loop(0, ROWS_PER_TILE, zrow_fill, 0)

  pltpu.sync_copy(zrow_v, acc_sh.at[pl.ds(s * ROWS_PER_TILE, ROWS_PER_TILE)])

  @pl.when(s == 0)
  def _():
    def zcnt_fill(i, carry):
      zcnt_v[pl.ds(i * 16, 16)] = z16
      return carry
    lax.fori_loop(0, N_NODES // 16, zcnt_fill, 0)
    pltpu.sync_copy(zcnt_v, cnt_sh)

  pltpu.sync_copy(idx_hbm.at[w], idx_v)

  plsc.subcore_barrier()

  ea_w = ea_hbm.at[w]

  def fire_stage(t, b):
    ds = []
    for j in range(WPS):
      idx_row = idx_v.at[t * WPS + j]
      ds.append(pltpu.async_copy(ea_v.at[b].at[pl.ds(j * SCAT, SCAT)],
                                 acc_sh.at[idx_row], scat_sem, add=True))
      ds.append(pltpu.async_copy(ones_v.at[pl.ds(0, SCAT)],
                                 cnt_sh.at[idx_row], scat_sem, add=True))
    return ds

  load = pltpu.async_copy(ea_w.at[pl.ds(0, STAGE)], ea_v.at[0], ld_sem)
  prev_scats = []
  for t in range(NSTAGE):
    b = t % 2
    load.wait()
    for d in prev_scats:
      d.wait()
    if t + 1 < NSTAGE:
      load = pltpu.async_copy(ea_w.at[pl.ds((t + 1) * STAGE, STAGE)],
                              ea_v.at[1 - b], ld_sem)
    prev_scats = fire_stage(t, b)
  for d in prev_scats:
    d.wait()

  plsc.subcore_barrier()

  @pl.when(s == 0)
  def _():
    pltpu.sync_copy(acc_sh, acc_out.at[c])

  @pl.when(jnp.logical_and(s == 0, c == 0))
  def _():
    pltpu.sync_copy(cnt_sh, cnt0_out)

  @pl.when(jnp.logical_and(s == 0, c == 1))
  def _():
    pltpu.sync_copy(cnt_sh, cnt1_out)


_sc_segment_sum = functools.partial(
    pl.kernel,
    out_type=(
        jax.ShapeDtypeStruct((NC, N_NODES, D_BOND), jnp.float32),
        jax.ShapeDtypeStruct((N_NODES,), jnp.float32),
        jax.ShapeDtypeStruct((N_NODES,), jnp.float32),
    ),
    mesh=plsc.VectorSubcoreMesh(core_axis_name="c", subcore_axis_name="s",
                                num_cores=NC, num_subcores=NS),
    compiler_params=pltpu.CompilerParams(use_tc_tiling_on_sc=False),
    scratch_types=[
        pltpu.VMEM((2, STAGE, D_BOND), jnp.float32),
        pltpu.VMEM((WPW, SCAT), jnp.int32),
        pltpu.VMEM((128,), jnp.float32),
        pltpu.VMEM((ROWS_PER_TILE, D_BOND), jnp.float32),
        pltpu.VMEM((N_NODES,), jnp.float32),
        pltpu.VMEM_SHARED((N_NODES, D_BOND), jnp.float32),
        pltpu.VMEM_SHARED((N_NODES,), jnp.float32),
        pltpu.SemaphoreType.DMA,
        pltpu.SemaphoreType.DMA,
    ],
)(_sc_body)


def _tc_body(p_ref, cnt_ref, x_ref, wa_ref, ba_ref, wb_ref, bb_ref,
             g_ref, be_ref, o_ref):
  agg = p_ref[0] + p_ref[1]
  h = lax.dot_general(x_ref[...], wa_ref[...],
                      (((1,), (1,)), ((), ())),
                      preferred_element_type=jnp.float32)
  h = h + lax.dot_general(agg, wb_ref[...],
                          (((1,), (1,)), ((), ())),
                          preferred_element_type=jnp.float32)
  bb2 = jnp.broadcast_to(bb_ref[...], (2, D_HID))
  h = h + lax.dot_general(cnt_ref[...], bb2,
                          (((1,), (0,)), ((), ())),
                          preferred_element_type=jnp.float32)
  h = h + ba_ref[...]
  h = jnp.maximum(h, 0.0)
  mean = jnp.mean(h, axis=0, keepdims=True)
  d = h - mean
  var = jnp.mean(d * d, axis=0, keepdims=True)
  o_ref[...] = g_ref[...] * (d * lax.rsqrt(var + EPS)) + be_ref[...]


_tc_dense = pl.pallas_call(
    _tc_body,
    out_shape=jax.ShapeDtypeStruct((N_NODES, D_HID), jnp.float32),
)


def kernel(x, edge_index, edge_attr, W_atom, b_atom, W_bond, b_bond,
           gamma, beta):
  dst = edge_index[1].astype(jnp.int32)
  idx3 = dst.reshape(NW, WPW, SCAT)
  ea3 = edge_attr.reshape(NW, EPW, D_BOND)
  partials, cnt0, cnt1 = _sc_segment_sum(ea3, idx3)
  counts2 = jnp.stack([cnt0, cnt1], axis=1)
  return _tc_dense(partials, counts2, x,
                   W_atom, b_atom.reshape(1, D_HID),
                   W_bond, b_bond.reshape(1, D_HID),
                   gamma.reshape(1, D_HID), beta.reshape(1, D_HID))


# speedup vs baseline: 5.8853x; 1.0009x over previous
import functools

import jax
import jax.numpy as jnp
from jax import lax
from jax.experimental import pallas as pl
from jax.experimental.pallas import tpu as pltpu
from jax.experimental.pallas import tpu_sc as plsc

N_NODES = 10000
N_EDGES = 320000
D_ATOM = 128
D_BOND = 16
D_HID = 128
EPS = 1e-5

NC = 2
NS = 16
NW = NC * NS
EPW = N_EDGES // NW
SCAT = 125
WPW = EPW // SCAT
STAGE = 2000
NSTAGE = EPW // STAGE
WPS = STAGE // SCAT
ROWS_PER_TILE = N_NODES // NS


def _sc_body(ea_hbm, idx_hbm, acc_out, cnt0_out, cnt1_out,
             ea_v, idx_v, ones_v, zrow_v, zcnt_v, acc_sh, cnt_sh,
             ld_sem, scat_sem):
  c = lax.axis_index("c")
  s = lax.axis_index("s")
  w = c * NS + s

  z16 = jnp.zeros((16,), jnp.float32)
  one16 = jnp.ones((16,), jnp.float32)

  for j in range(128 // 16):
    ones_v[pl.ds(j * 16, 16)] = one16

  def zrow_fill(i, carry):
    zrow_v[i, :] = z16
    return carry
  lax.fori_loop(0, ROWS_PER_TILE, zrow_fill, 0)

  pltpu.sync_copy(zrow_v, acc_sh.at[pl.ds(s * ROWS_PER_TILE, ROWS_PER_TILE)])

  @pl.when(s == 0)
  def _():
    def zcnt_fill(i, carry):
      zcnt_v[pl.ds(i * 16, 16)] = z16
      return carry
    lax.fori_loop(0, N_NODES // 16, zcnt_fill, 0)
    pltpu.sync_copy(zcnt_v, cnt_sh)

  pltpu.sync_copy(idx_hbm.at[w], idx_v)

  plsc.subcore_barrier()

  ea_w = ea_hbm.at[w]

  def fire_stage(t, b):
    ds = []
    for j in range(WPS):
      idx_row = idx_v.at[t * WPS + j]
      ds.append(pltpu.async_copy(ea_v.at[b].at[pl.ds(j * SCAT, SCAT)],
                                 acc_sh.at[idx_row], scat_sem, add=True))
      ds.append(pltpu.async_copy(ones_v.at[pl.ds(0, SCAT)],
                                 cnt_sh.at[idx_row], scat_sem, add=True))
    return ds

  load = pltpu.async_copy(ea_w.at[pl.ds(0, STAGE)], ea_v.at[0], ld_sem)
  prev_scats = []
  for t in range(NSTAGE):
    b = t % 2
    load.wait()
    for d in prev_scats:
      d.wait()
    if t + 1 < NSTAGE:
      load = pltpu.async_copy(ea_w.at[pl.ds((t + 1) * STAGE, STAGE)],
                              ea_v.at[1 - b], ld_sem)
    prev_scats = fire_stage(t, b)
  for d in prev_scats:
    d.wait()

  plsc.subcore_barrier()

  @pl.when(s == 0)
  def _():
    pltpu.sync_copy(acc_sh, acc_out.at[c])

  @pl.when(jnp.logical_and(s == 0, c == 0))
  def _():
    pltpu.sync_copy(cnt_sh, cnt0_out)

  @pl.when(jnp.logical_and(s == 0, c == 1))
  def _():
    pltpu.sync_copy(cnt_sh, cnt1_out)


_sc_segment_sum = functools.partial(
    pl.kernel,
    out_type=(
        jax.ShapeDtypeStruct((NC, N_NODES, D_BOND), jnp.float32),
        jax.ShapeDtypeStruct((N_NODES,), jnp.float32),
        jax.ShapeDtypeStruct((N_NODES,), jnp.float32),
    ),
    mesh=plsc.VectorSubcoreMesh(core_axis_name="c", subcore_axis_name="s",
                                num_cores=NC, num_subcores=NS),
    compiler_params=pltpu.CompilerParams(use_tc_tiling_on_sc=False),
    scratch_types=[
        pltpu.VMEM((2, STAGE, D_BOND), jnp.float32),
        pltpu.VMEM((WPW, SCAT), jnp.int32),
        pltpu.VMEM((128,), jnp.float32),
        pltpu.VMEM((ROWS_PER_TILE, D_BOND), jnp.float32),
        pltpu.VMEM((N_NODES,), jnp.float32),
        pltpu.VMEM_SHARED((N_NODES, D_BOND), jnp.float32),
        pltpu.VMEM_SHARED((N_NODES,), jnp.float32),
        pltpu.SemaphoreType.DMA,
        pltpu.SemaphoreType.DMA,
    ],
)(_sc_body)


def _tc_pre_body(x_ref, wa_ref, ba_ref, o_ref):
  o_ref[...] = lax.dot_general(x_ref[...], wa_ref[...],
                               (((1,), (1,)), ((), ())),
                               preferred_element_type=jnp.float32
                               ) + ba_ref[...]


_tc_pre = pl.pallas_call(
    _tc_pre_body,
    out_shape=jax.ShapeDtypeStruct((N_NODES, D_HID), jnp.float32),
)


def _tc_post_body(xa_ref, p_ref, cnt_ref, wb_ref, bb_ref,
                  g_ref, be_ref, o_ref):
  agg = p_ref[0] + p_ref[1]
  h = xa_ref[...] + lax.dot_general(agg, wb_ref[...],
                                    (((1,), (1,)), ((), ())),
                                    preferred_element_type=jnp.float32)
  bb2 = jnp.broadcast_to(bb_ref[...], (2, D_HID))
  h = h + lax.dot_general(cnt_ref[...], bb2,
                          (((1,), (0,)), ((), ())),
                          preferred_element_type=jnp.float32)
  h = jnp.maximum(h, 0.0)
  mean = jnp.mean(h, axis=0, keepdims=True)
  d = h - mean
  var = jnp.mean(d * d, axis=0, keepdims=True)
  o_ref[...] = g_ref[...] * (d * lax.rsqrt(var + EPS)) + be_ref[...]


_tc_post = pl.pallas_call(
    _tc_post_body,
    out_shape=jax.ShapeDtypeStruct((N_NODES, D_HID), jnp.float32),
)


def kernel(x, edge_index, edge_attr, W_atom, b_atom, W_bond, b_bond,
           gamma, beta):
  dst = edge_index[1].astype(jnp.int32)
  idx3 = dst.reshape(NW, WPW, SCAT)
  ea3 = edge_attr.reshape(NW, EPW, D_BOND)
  partials, cnt0, cnt1 = _sc_segment_sum(ea3, idx3)
  xa = _tc_pre(x, W_atom, b_atom.reshape(1, D_HID))
  counts2 = jnp.stack([cnt0, cnt1], axis=1)
  return _tc_post(xa, partials, counts2,
                  W_bond, b_bond.reshape(1, D_HID),
                  gamma.reshape(1, D_HID), beta.reshape(1, D_HID))


# speedup vs baseline: 5.9713x; 1.0146x over previous
import functools

import jax
import jax.numpy as jnp
from jax import lax
from jax.experimental import pallas as pl
from jax.experimental.pallas import tpu as pltpu
from jax.experimental.pallas import tpu_sc as plsc

N_NODES = 10000
N_EDGES = 320000
D_ATOM = 128
D_BOND = 16
D_HID = 128
EPS = 1e-5

NC = 2
NS = 16
NW = NC * NS
EPW = N_EDGES // NW
SCAT = 125
WPW = EPW // SCAT
STAGE = 2000
NSTAGE = EPW // STAGE
WPS = STAGE // SCAT
ROWS_PER_TILE = N_NODES // NS


def _sc_body(ea_hbm, idx_hbm, acc_out, cnt0_out, cnt1_out,
             ea_v, idx_v, ones_v, zrow_v, zcnt_v, acc_sh, cnt_sh,
             ld_sem, idx_sem, scat_sem):
  c = lax.axis_index("c")
  s = lax.axis_index("s")
  w = c * NS + s

  idx_ld = pltpu.async_copy(idx_hbm.at[w], idx_v, idx_sem)
  ea_w = ea_hbm.at[w]
  load = pltpu.async_copy(ea_w.at[pl.ds(0, STAGE)], ea_v.at[0], ld_sem)

  z16 = jnp.zeros((16,), jnp.float32)
  one16 = jnp.ones((16,), jnp.float32)

  for j in range(128 // 16):
    ones_v[pl.ds(j * 16, 16)] = one16

  def zrow_fill(i, carry):
    zrow_v[i, :] = z16
    return carry
  lax.fori_loop(0, ROWS_PER_TILE, zrow_fill, 0)

  pltpu.sync_copy(zrow_v, acc_sh.at[pl.ds(s * ROWS_PER_TILE, ROWS_PER_TILE)])

  @pl.when(s == 0)
  def _():
    def zcnt_fill(i, carry):
      zcnt_v[pl.ds(i * 16, 16)] = z16
      return carry
    lax.fori_loop(0, N_NODES // 16, zcnt_fill, 0)
    pltpu.sync_copy(zcnt_v, cnt_sh)

  idx_ld.wait()
  plsc.subcore_barrier()

  def fire_stage(t, b):
    ds = []
    for j in range(WPS):
      idx_row = idx_v.at[t * WPS + j]
      ds.append(pltpu.async_copy(ea_v.at[b].at[pl.ds(j * SCAT, SCAT)],
                                 acc_sh.at[idx_row], scat_sem, add=True))
      ds.append(pltpu.async_copy(ones_v.at[pl.ds(0, SCAT)],
                                 cnt_sh.at[idx_row], scat_sem, add=True))
    return ds

  prev_scats = []
  for t in range(NSTAGE):
    b = t % 2
    load.wait()
    for d in prev_scats:
      d.wait()
    if t + 1 < NSTAGE:
      load = pltpu.async_copy(ea_w.at[pl.ds((t + 1) * STAGE, STAGE)],
                              ea_v.at[1 - b], ld_sem)
    prev_scats = fire_stage(t, b)
  for d in prev_scats:
    d.wait()

  plsc.subcore_barrier()

  pltpu.sync_copy(acc_sh.at[pl.ds(s * ROWS_PER_TILE, ROWS_PER_TILE)],
                  acc_out.at[c].at[pl.ds(s * ROWS_PER_TILE, ROWS_PER_TILE)])

  @pl.when(jnp.logical_and(s == 0, c == 0))
  def _():
    pltpu.sync_copy(cnt_sh, cnt0_out)

  @pl.when(jnp.logical_and(s == 0, c == 1))
  def _():
    pltpu.sync_copy(cnt_sh, cnt1_out)


_sc_segment_sum = functools.partial(
    pl.kernel,
    out_type=(
        jax.ShapeDtypeStruct((NC, N_NODES, D_BOND), jnp.float32),
        jax.ShapeDtypeStruct((N_NODES,), jnp.float32),
        jax.ShapeDtypeStruct((N_NODES,), jnp.float32),
    ),
    mesh=plsc.VectorSubcoreMesh(core_axis_name="c", subcore_axis_name="s",
                                num_cores=NC, num_subcores=NS),
    compiler_params=pltpu.CompilerParams(use_tc_tiling_on_sc=False,
                                         skip_device_barrier=True),
    scratch_types=[
        pltpu.VMEM((2, STAGE, D_BOND), jnp.float32),
        pltpu.VMEM((WPW, SCAT), jnp.int32),
        pltpu.VMEM((128,), jnp.float32),
        pltpu.VMEM((ROWS_PER_TILE, D_BOND), jnp.float32),
        pltpu.VMEM((N_NODES,), jnp.float32),
        pltpu.VMEM_SHARED((N_NODES, D_BOND), jnp.float32),
        pltpu.VMEM_SHARED((N_NODES,), jnp.float32),
        pltpu.SemaphoreType.DMA,
        pltpu.SemaphoreType.DMA,
        pltpu.SemaphoreType.DMA,
    ],
)(_sc_body)


def _tc_pre_body(x_ref, wa_ref, ba_ref, o_ref):
  o_ref[...] = lax.dot_general(x_ref[...], wa_ref[...],
                               (((1,), (1,)), ((), ())),
                               preferred_element_type=jnp.float32
                               ) + ba_ref[...]


_tc_pre = pl.pallas_call(
    _tc_pre_body,
    out_shape=jax.ShapeDtypeStruct((N_NODES, D_HID), jnp.float32),
    compiler_params=pltpu.CompilerParams(skip_device_barrier=True),
)


def _tc_post_body(xa_ref, p_ref, cnt_ref, wb_ref, bb_ref,
                  g_ref, be_ref, o_ref):
  agg = p_ref[0] + p_ref[1]
  h = xa_ref[...] + lax.dot_general(agg, wb_ref[...],
                                    (((1,), (1,)), ((), ())),
                                    preferred_element_type=jnp.float32)
  bb2 = jnp.broadcast_to(bb_ref[...], (2, D_HID))
  h = h + lax.dot_general(cnt_ref[...], bb2,
                          (((1,), (0,)), ((), ())),
                          preferred_element_type=jnp.float32)
  h = jnp.maximum(h, 0.0)
  mean = jnp.mean(h, axis=0, keepdims=True)
  d = h - mean
  var = jnp.mean(d * d, axis=0, keepdims=True)
  o_ref[...] = g_ref[...] * (d * lax.rsqrt(var + EPS)) + be_ref[...]


_tc_post = pl.pallas_call(
    _tc_post_body,
    out_shape=jax.ShapeDtypeStruct((N_NODES, D_HID), jnp.float32),
    compiler_params=pltpu.CompilerParams(skip_device_barrier=True),
)


def kernel(x, edge_index, edge_attr, W_atom, b_atom, W_bond, b_bond,
           gamma, beta):
  dst = edge_index[1].astype(jnp.int32)
  idx3 = dst.reshape(NW, WPW, SCAT)
  ea3 = edge_attr.reshape(NW, EPW, D_BOND)
  partials, cnt0, cnt1 = _sc_segment_sum(ea3, idx3)
  xa = _tc_pre(x, W_atom, b_atom.reshape(1, D_HID))
  counts2 = jnp.stack([cnt0, cnt1], axis=1)
  return _tc_post(xa, partials, counts2,
                  W_bond, b_bond.reshape(1, D_HID),
                  gamma.reshape(1, D_HID), beta.reshape(1, D_HID))


# speedup vs baseline: 6.1466x; 1.0294x over previous
import functools

import jax
import jax.numpy as jnp
from jax import lax
from jax.experimental import pallas as pl
from jax.experimental.pallas import tpu as pltpu
from jax.experimental.pallas import tpu_sc as plsc

N_NODES = 10000
N_EDGES = 320000
D_ATOM = 128
D_BOND = 16
D_HID = 128
EPS = 1e-5

NC = 2
NS = 16
NW = NC * NS
EPW = N_EDGES // NW
SCAT = 125
WPW = EPW // SCAT
STAGE = 2000
NSTAGE = EPW // STAGE
WPS = STAGE // SCAT
ROWS_PER_TILE = N_NODES // NS


def _sc_body(ea_hbm, idx_hbm, acc_out, cnt_out,
             ea_v, idx_v, ones_v, zrow_v, zcnt_v, acc_sh, cnt_sh,
             ld_sem, idx_sem, scat_sem):
  c = lax.axis_index("c")
  s = lax.axis_index("s")
  w = c * NS + s

  idx_ld = pltpu.async_copy(idx_hbm.at[w], idx_v, idx_sem)
  ea_w = ea_hbm.at[w]
  load = pltpu.async_copy(ea_w.at[pl.ds(0, STAGE)], ea_v.at[0], ld_sem)

  z16 = jnp.zeros((16,), jnp.float32)
  one16 = jnp.ones((16,), jnp.float32)

  for j in range(128 // 16):
    ones_v[pl.ds(j * 16, 16)] = one16

  def zrow_fill(i, carry):
    zrow_v[i, :] = z16
    return carry
  lax.fori_loop(0, ROWS_PER_TILE, zrow_fill, 0)

  pltpu.sync_copy(zrow_v, acc_sh.at[pl.ds(s * ROWS_PER_TILE, ROWS_PER_TILE)])

  @pl.when(s == 0)
  def _():
    def zcnt_fill(i, carry):
      zcnt_v[pl.ds(i * 16, 16)] = z16
      return carry
    lax.fori_loop(0, N_NODES // 16, zcnt_fill, 0)
    pltpu.sync_copy(zcnt_v, cnt_sh)

  idx_ld.wait()
  plsc.subcore_barrier()

  def fire_stage(t, b):
    ds = []
    for j in range(WPS):
      idx_row = idx_v.at[t * WPS + j]
      ds.append(pltpu.async_copy(ea_v.at[b].at[pl.ds(j * SCAT, SCAT)],
                                 acc_sh.at[idx_row], scat_sem, add=True))
      ds.append(pltpu.async_copy(ones_v.at[pl.ds(0, SCAT)],
                                 cnt_sh.at[idx_row], scat_sem, add=True))
    return ds

  prev_scats = []
  for t in range(NSTAGE):
    b = t % 2
    load.wait()
    for d in prev_scats:
      d.wait()
    if t + 1 < NSTAGE:
      load = pltpu.async_copy(ea_w.at[pl.ds((t + 1) * STAGE, STAGE)],
                              ea_v.at[1 - b], ld_sem)
    prev_scats = fire_stage(t, b)
  for d in prev_scats:
    d.wait()

  plsc.subcore_barrier()

  pltpu.sync_copy(acc_sh.at[pl.ds(s * ROWS_PER_TILE, ROWS_PER_TILE)],
                  acc_out.at[c].at[pl.ds(s * ROWS_PER_TILE, ROWS_PER_TILE)])

  @pl.when(s == 0)
  def _():
    pltpu.sync_copy(cnt_sh, cnt_out.at[c])


_sc_segment_sum = functools.partial(
    pl.kernel,
    out_type=(
        jax.ShapeDtypeStruct((NC, N_NODES, D_BOND), jnp.float32),
        jax.ShapeDtypeStruct((NC, N_NODES), jnp.float32),
    ),
    mesh=plsc.VectorSubcoreMesh(core_axis_name="c", subcore_axis_name="s",
                                num_cores=NC, num_subcores=NS),
    compiler_params=pltpu.CompilerParams(use_tc_tiling_on_sc=False,
                                         skip_device_barrier=True),
    scratch_types=[
        pltpu.VMEM((2, STAGE, D_BOND), jnp.float32),
        pltpu.VMEM((WPW, SCAT), jnp.int32),
        pltpu.VMEM((128,), jnp.float32),
        pltpu.VMEM((ROWS_PER_TILE, D_BOND), jnp.float32),
        pltpu.VMEM((N_NODES,), jnp.float32),
        pltpu.VMEM_SHARED((N_NODES, D_BOND), jnp.float32),
        pltpu.VMEM_SHARED((N_NODES,), jnp.float32),
        pltpu.SemaphoreType.DMA,
        pltpu.SemaphoreType.DMA,
        pltpu.SemaphoreType.DMA,
    ],
)(_sc_body)


def _tc_body(x_ref, wa_ref, ba_ref, p_ref, cnt_ref, wb_ref, bb_ref,
             g_ref, be_ref, o_ref):
  h = lax.dot_general(x_ref[...], wa_ref[...],
                      (((1,), (1,)), ((), ())),
                      preferred_element_type=jnp.float32) + ba_ref[...]
  agg = p_ref[0] + p_ref[1]
  h = h + lax.dot_general(agg, wb_ref[...],
                          (((1,), (1,)), ((), ())),
                          preferred_element_type=jnp.float32)
  bb2 = jnp.broadcast_to(bb_ref[...], (2, D_HID))
  h = h + lax.dot_general(cnt_ref[...], bb2,
                          (((0,), (0,)), ((), ())),
                          preferred_element_type=jnp.float32)
  h = jnp.maximum(h, 0.0)
  mean = jnp.mean(h, axis=0, keepdims=True)
  d = h - mean
  var = jnp.mean(d * d, axis=0, keepdims=True)
  o_ref[...] = g_ref[...] * (d * lax.rsqrt(var + EPS)) + be_ref[...]


_tc_dense = pl.pallas_call(
    _tc_body,
    out_shape=jax.ShapeDtypeStruct((N_NODES, D_HID), jnp.float32),
    compiler_params=pltpu.CompilerParams(skip_device_barrier=True),
)


def kernel(x, edge_index, edge_attr, W_atom, b_atom, W_bond, b_bond,
           gamma, beta):
  dst = edge_index[1].astype(jnp.int32)
  idx3 = dst.reshape(NW, WPW, SCAT)
  ea3 = edge_attr.reshape(NW, EPW, D_BOND)
  partials, counts = _sc_segment_sum(ea3, idx3)
  return _tc_dense(x, W_atom, b_atom.reshape(1, D_HID),
                   partials, counts,
                   W_bond, b_bond.reshape(1, D_HID),
                   gamma.reshape(1, D_HID), beta.reshape(1, D_HID))


# speedup vs baseline: 6.1697x; 1.0037x over previous
import functools

import jax
import jax.numpy as jnp
from jax import lax
from jax.experimental import pallas as pl
from jax.experimental.pallas import tpu as pltpu
from jax.experimental.pallas import tpu_sc as plsc

N_NODES = 10000
N_EDGES = 320000
D_ATOM = 128
D_BOND = 16
D_HID = 128
EPS = 1e-5

NC = 2
NS = 16
NW = NC * NS
EPW = N_EDGES // NW
SCAT = 125
WPW = EPW // SCAT
STAGE = 2500
NSTAGE = EPW // STAGE
WPS = STAGE // SCAT
ROWS_PER_TILE = N_NODES // NS


def _sc_body(ea_hbm, idx_hbm, acc_out, cnt_out,
             ea_v, idx_v, ones_v, zrow_v, zcnt_v, acc_sh, cnt_sh,
             ld_sem, idx_sem, scat_sem):
  c = lax.axis_index("c")
  s = lax.axis_index("s")
  w = c * NS + s

  idx_ld = pltpu.async_copy(idx_hbm.at[w], idx_v, idx_sem)
  ea_w = ea_hbm.at[w]
  load = pltpu.async_copy(ea_w.at[pl.ds(0, STAGE)], ea_v.at[0], ld_sem)

  z16 = jnp.zeros((16,), jnp.float32)
  one16 = jnp.ones((16,), jnp.float32)

  for j in range(128 // 16):
    ones_v[pl.ds(j * 16, 16)] = one16

  def zrow_fill(i, carry):
    zrow_v[i, :] = z16
    return carry
  lax.fori_loop(0, ROWS_PER_TILE, zrow_fill, 0)

  pltpu.sync_copy(zrow_v, acc_sh.at[pl.ds(s * ROWS_PER_TILE, ROWS_PER_TILE)])

  @pl.when(s == 0)
  def _():
    def zcnt_fill(i, carry):
      zcnt_v[pl.ds(i * 16, 16)] = z16
      return carry
    lax.fori_loop(0, N_NODES // 16, zcnt_fill, 0)
    pltpu.sync_copy(zcnt_v, cnt_sh)

  idx_ld.wait()
  plsc.subcore_barrier()

  def fire_stage(t, b):
    ds = []
    for j in range(WPS):
      idx_row = idx_v.at[t * WPS + j]
      ds.append(pltpu.async_copy(ea_v.at[b].at[pl.ds(j * SCAT, SCAT)],
                                 acc_sh.at[idx_row], scat_sem, add=True))
      ds.append(pltpu.async_copy(ones_v.at[pl.ds(0, SCAT)],
                                 cnt_sh.at[idx_row], scat_sem, add=True))
    return ds

  prev_scats = []
  for t in range(NSTAGE):
    b = t % 2
    load.wait()
    for d in prev_scats:
      d.wait()
    if t + 1 < NSTAGE:
      load = pltpu.async_copy(ea_w.at[pl.ds((t + 1) * STAGE, STAGE)],
                              ea_v.at[1 - b], ld_sem)
    prev_scats = fire_stage(t, b)
  for d in prev_scats:
    d.wait()

  plsc.subcore_barrier()

  pltpu.sync_copy(acc_sh.at[pl.ds(s * ROWS_PER_TILE, ROWS_PER_TILE)],
                  acc_out.at[c].at[pl.ds(s * ROWS_PER_TILE, ROWS_PER_TILE)])

  @pl.when(s == 0)
  def _():
    pltpu.sync_copy(cnt_sh, cnt_out.at[c])


_sc_segment_sum = functools.partial(
    pl.kernel,
    out_type=(
        jax.ShapeDtypeStruct((NC, N_NODES, D_BOND), jnp.float32),
        jax.ShapeDtypeStruct((NC, N_NODES), jnp.float32),
    ),
    mesh=plsc.VectorSubcoreMesh(core_axis_name="c", subcore_axis_name="s",
                                num_cores=NC, num_subcores=NS),
    compiler_params=pltpu.CompilerParams(use_tc_tiling_on_sc=False,
                                         skip_device_barrier=True),
    scratch_types=[
        pltpu.VMEM((2, STAGE, D_BOND), jnp.float32),
        pltpu.VMEM((WPW, SCAT), jnp.int32),
        pltpu.VMEM((128,), jnp.float32),
        pltpu.VMEM((ROWS_PER_TILE, D_BOND), jnp.float32),
        pltpu.VMEM((N_NODES,), jnp.float32),
        pltpu.VMEM_SHARED((N_NODES, D_BOND), jnp.float32),
        pltpu.VMEM_SHARED((N_NODES,), jnp.float32),
        pltpu.SemaphoreType.DMA,
        pltpu.SemaphoreType.DMA,
        pltpu.SemaphoreType.DMA,
    ],
)(_sc_body)


def _tc_body(x_ref, wa_ref, ba_ref, p_ref, cnt_ref, wb_ref, bb_ref,
             g_ref, be_ref, o_ref):
  h = lax.dot_general(x_ref[...], wa_ref[...],
                      (((1,), (1,)), ((), ())),
                      preferred_element_type=jnp.float32) + ba_ref[...]
  agg = p_ref[0] + p_ref[1]
  h = h + lax.dot_general(agg, wb_ref[...],
                          (((1,), (1,)), ((), ())),
                          preferred_element_type=jnp.float32)
  bb2 = jnp.broadcast_to(bb_ref[...], (2, D_HID))
  h = h + lax.dot_general(cnt_ref[...], bb2,
                          (((0,), (0,)), ((), ())),
                          preferred_element_type=jnp.float32)
  h = jnp.maximum(h, 0.0)
  mean = jnp.mean(h, axis=0, keepdims=True)
  d = h - mean
  var = jnp.mean(d * d, axis=0, keepdims=True)
  o_ref[...] = g_ref[...] * (d * lax.rsqrt(var + EPS)) + be_ref[...]


_tc_dense = pl.pallas_call(
    _tc_body,
    out_shape=jax.ShapeDtypeStruct((N_NODES, D_HID), jnp.float32),
    compiler_params=pltpu.CompilerParams(skip_device_barrier=True),
)


def kernel(x, edge_index, edge_attr, W_atom, b_atom, W_bond, b_bond,
           gamma, beta):
  dst = edge_index[1].astype(jnp.int32)
  idx3 = dst.reshape(NW, WPW, SCAT)
  ea3 = edge_attr.reshape(NW, EPW, D_BOND)
  partials, counts = _sc_segment_sum(ea3, idx3)
  return _tc_dense(x, W_atom, b_atom.reshape(1, D_HID),
                   partials, counts,
                   W_bond, b_bond.reshape(1, D_HID),
                   gamma.reshape(1, D_HID), beta.reshape(1, D_HID))
